# Initial kernel scaffold; baseline (speedup 1.0000x reference)
import jax
import jax.numpy as jnp
from jax.experimental import pallas as pl


def kernel(x_global, seqlens, seqlens_perm_idxs, chunk_sel):
    raise NotImplementedError("write your pallas kernel here")


# speedup vs baseline: 5.3343x; 5.3343x over previous
import functools

import jax
import jax.numpy as jnp
from jax import lax
from jax.experimental import pallas as pl
from jax.experimental.pallas import tpu as pltpu
from jax.experimental.pallas import tpu_sc as plsc

TOTAL = 16384
D_MODEL = 2048
NUM_CHUNKS = 64
CHUNK = TOTAL // NUM_CHUNKS
NSEL = 8
OUT_ROWS = NSEL * CHUNK
NC, NS, L = 2, 16, 16
NW = NC * NS
ROWS_PER_W = OUT_ROWS // NW
STAGE = 16
NSTAGES = ROWS_PER_W // STAGE
NVEC = ROWS_PER_W // L


def _cumsum8(vec, iota):
    acc = jnp.zeros((L,), jnp.int32)
    for s in range(NSEL):
        acc = acc + jnp.where(iota >= s, vec[s], 0)
    return acc


def _body(x_hbm, meta_hbm, out_hbm,
          meta_v, starts_v, adj_v, idx_v,
          buf0, buf1, sem_in0, sem_in1, sem_out0, sem_out1):
    wid = lax.axis_index("s") * NC + lax.axis_index("c")
    base = wid * ROWS_PER_W
    iota = lax.iota(jnp.int32, L)

    pltpu.sync_copy(meta_hbm, meta_v)
    seql = meta_v[pl.ds(0, L)]
    perm = meta_v[pl.ds(L, L)]
    starts = _cumsum8(seql, iota) - seql
    starts_v[...] = starts
    slp = plsc.load_gather(meta_v, [perm])
    ends = _cumsum8(slp, iota)
    adj_v[...] = plsc.load_gather(starts_v, [perm]) - (ends - slp)

    for v in range(NVEC):
        t_out = base + (v * L) + iota
        c = lax.shift_right_logical(t_out, 8)
        within = jnp.bitwise_and(t_out, CHUNK - 1)
        selc = plsc.load_gather(meta_v, [c + 2 * L])
        t = lax.shift_left(selc, 8) + within
        j = jnp.zeros((L,), jnp.int32)
        for s in range(NSEL):
            j += jnp.where(t >= ends[s], 1, 0).astype(jnp.int32)
        idx_v[pl.ds(v * L, L)] = t + plsc.load_gather(adj_v, [j])

    bufs = (buf0, buf1)
    sems_in = (sem_in0, sem_in1)
    sems_out = (sem_out0, sem_out1)
    cp_in = [None] * NSTAGES
    cp_out = [None] * NSTAGES
    cp_in[0] = pltpu.async_copy(
        x_hbm.at[idx_v.at[pl.ds(0, STAGE)]], bufs[0], sems_in[0])
    for s in range(NSTAGES):
        b = s % 2
        if s + 1 < NSTAGES:
            if s - 1 >= 0:
                cp_out[s - 1].wait()
            cp_in[s + 1] = pltpu.async_copy(
                x_hbm.at[idx_v.at[pl.ds((s + 1) * STAGE, STAGE)]],
                bufs[1 - b], sems_in[1 - b])
        cp_in[s].wait()
        cp_out[s] = pltpu.async_copy(
            bufs[b], out_hbm.at[pl.ds(base + s * STAGE, STAGE)], sems_out[b])
    cp_out[NSTAGES - 2].wait()
    cp_out[NSTAGES - 1].wait()


@jax.jit
def _dispatch(x_global, meta):
    mesh = plsc.VectorSubcoreMesh(core_axis_name="c", subcore_axis_name="s")
    run = functools.partial(
        pl.kernel,
        mesh=mesh,
        compiler_params=pltpu.CompilerParams(needs_layout_passes=False),
        out_type=jax.ShapeDtypeStruct((OUT_ROWS, D_MODEL), jnp.float32),
        scratch_types=[
            pltpu.VMEM((3 * L,), jnp.int32),
            pltpu.VMEM((L,), jnp.int32),
            pltpu.VMEM((L,), jnp.int32),
            pltpu.VMEM((ROWS_PER_W,), jnp.int32),
            pltpu.VMEM((STAGE, D_MODEL), jnp.float32),
            pltpu.VMEM((STAGE, D_MODEL), jnp.float32),
            pltpu.SemaphoreType.DMA,
            pltpu.SemaphoreType.DMA,
            pltpu.SemaphoreType.DMA,
            pltpu.SemaphoreType.DMA,
        ],
    )(_body)
    return run(x_global, meta)


def kernel(x_global, seqlens, seqlens_perm_idxs, chunk_sel):
    seqlens = jnp.asarray(seqlens, jnp.int32)
    perm = jnp.asarray(seqlens_perm_idxs, jnp.int32)
    sel = jnp.asarray(chunk_sel, jnp.int32)
    meta = (jnp.zeros((3 * L,), jnp.int32)
            .at[0:NSEL].set(seqlens)
            .at[L:L + NSEL].set(perm)
            .at[2 * L:2 * L + NSEL].set(sel))
    return _dispatch(x_global, meta)


# speedup vs baseline: 5.5643x; 1.0431x over previous
import functools

import jax
import jax.numpy as jnp
from jax import lax
from jax.experimental import pallas as pl
from jax.experimental.pallas import tpu as pltpu
from jax.experimental.pallas import tpu_sc as plsc

TOTAL = 16384
D_MODEL = 2048
NUM_CHUNKS = 64
CHUNK = TOTAL // NUM_CHUNKS
NSEL = 8
OUT_ROWS = NSEL * CHUNK
NC, NS, L = 2, 16, 16
NW = NC * NS
ROWS_PER_W = OUT_ROWS // NW
STAGE = 16
NSTAGES = ROWS_PER_W // STAGE
NBUF = 3
NVEC = ROWS_PER_W // L


def _cumsum8(vec, iota):
    acc = jnp.zeros((L,), jnp.int32)
    for s in range(NSEL):
        acc = acc + jnp.where(iota >= s, vec[s], 0)
    return acc


def _body(x_hbm, meta_hbm, out_hbm,
          meta_v, starts_v, adj_v, idx_v,
          buf0, buf1, buf2,
          sem_in0, sem_in1, sem_in2, sem_out0, sem_out1, sem_out2):
    wid = lax.axis_index("s") * NC + lax.axis_index("c")
    base = wid * ROWS_PER_W
    iota = lax.iota(jnp.int32, L)

    pltpu.sync_copy(meta_hbm, meta_v)
    seql = meta_v[pl.ds(0, L)]
    perm = meta_v[pl.ds(L, L)]
    starts = _cumsum8(seql, iota) - seql
    starts_v[...] = starts
    slp = plsc.load_gather(meta_v, [perm])
    ends = _cumsum8(slp, iota)
    adj_v[...] = plsc.load_gather(starts_v, [perm]) - (ends - slp)

    for v in range(NVEC):
        t_out = base + (v * L) + iota
        c = lax.shift_right_logical(t_out, 8)
        within = jnp.bitwise_and(t_out, CHUNK - 1)
        selc = plsc.load_gather(meta_v, [c + 2 * L])
        t = lax.shift_left(selc, 8) + within
        j = jnp.zeros((L,), jnp.int32)
        for s in range(NSEL):
            j += jnp.where(t >= ends[s], 1, 0).astype(jnp.int32)
        idx_v[pl.ds(v * L, L)] = t + plsc.load_gather(adj_v, [j])

    bufs = (buf0, buf1, buf2)
    sems_in = (sem_in0, sem_in1, sem_in2)
    sems_out = (sem_out0, sem_out1, sem_out2)
    cp_in = [None] * NSTAGES
    cp_out = [None] * NSTAGES
    for s in range(min(NBUF, NSTAGES)):
        cp_in[s] = pltpu.async_copy(
            x_hbm.at[idx_v.at[pl.ds(s * STAGE, STAGE)]],
            bufs[s], sems_in[s])
    out_waited = [False] * NSTAGES
    for s in range(NSTAGES):
        b = s % NBUF
        cp_in[s].wait()
        cp_out[s] = pltpu.async_copy(
            bufs[b], out_hbm.at[pl.ds(base + s * STAGE, STAGE)], sems_out[b])
        nxt = s + NBUF
        if nxt < NSTAGES:
            cp_out[s].wait()
            out_waited[s] = True
            cp_in[nxt] = pltpu.async_copy(
                x_hbm.at[idx_v.at[pl.ds(nxt * STAGE, STAGE)]],
                bufs[b], sems_in[b])
    for s in range(NSTAGES):
        if not out_waited[s]:
            cp_out[s].wait()


@jax.jit
def _dispatch(x_global, meta):
    mesh = plsc.VectorSubcoreMesh(core_axis_name="c", subcore_axis_name="s")
    run = functools.partial(
        pl.kernel,
        mesh=mesh,
        compiler_params=pltpu.CompilerParams(needs_layout_passes=False),
        out_type=jax.ShapeDtypeStruct((OUT_ROWS, D_MODEL), jnp.float32),
        scratch_types=[
            pltpu.VMEM((3 * L,), jnp.int32),
            pltpu.VMEM((L,), jnp.int32),
            pltpu.VMEM((L,), jnp.int32),
            pltpu.VMEM((ROWS_PER_W,), jnp.int32),
            pltpu.VMEM((STAGE, D_MODEL), jnp.float32),
            pltpu.VMEM((STAGE, D_MODEL), jnp.float32),
            pltpu.VMEM((STAGE, D_MODEL), jnp.float32),
            pltpu.SemaphoreType.DMA,
            pltpu.SemaphoreType.DMA,
            pltpu.SemaphoreType.DMA,
            pltpu.SemaphoreType.DMA,
            pltpu.SemaphoreType.DMA,
            pltpu.SemaphoreType.DMA,
        ],
    )(_body)
    return run(x_global, meta)


def kernel(x_global, seqlens, seqlens_perm_idxs, chunk_sel):
    seqlens = jnp.asarray(seqlens, jnp.int32)
    perm = jnp.asarray(seqlens_perm_idxs, jnp.int32)
    sel = jnp.asarray(chunk_sel, jnp.int32)
    meta = (jnp.zeros((3 * L,), jnp.int32)
            .at[0:NSEL].set(seqlens)
            .at[L:L + NSEL].set(perm)
            .at[2 * L:2 * L + NSEL].set(sel))
    return _dispatch(x_global, meta)
